# Initial kernel scaffold; baseline (speedup 1.0000x reference)
import jax
import jax.numpy as jnp
from jax.experimental import pallas as pl


def kernel(z, pos_embed):
    raise NotImplementedError("write your pallas kernel here")


# speedup vs baseline: 1.2973x; 1.2973x over previous
import jax
import jax.numpy as jnp
from jax.experimental import pallas as pl

_SEQ = 2048
_D = 256
_S_BLK = 32


def _bcast_body(pe_ref, out_ref):
    pe = pe_ref[...]
    out_ref[...] = jnp.broadcast_to(pe[:, None, :], out_ref.shape)


def kernel(z, pos_embed):
    batch = z.shape[1]
    out = pl.pallas_call(
        _bcast_body,
        grid=(_SEQ // _S_BLK,),
        in_specs=[pl.BlockSpec((_S_BLK, _D), lambda i: (i, 0))],
        out_specs=pl.BlockSpec((_S_BLK, batch, _D), lambda i: (i, 0, 0)),
        out_shape=jax.ShapeDtypeStruct((_SEQ, batch, _D), z.dtype),
    )(pos_embed)
    return out
